# Initial kernel scaffold; baseline (speedup 1.0000x reference)
import jax
import jax.numpy as jnp
from jax.experimental import pallas as pl


def kernel(features, edge_index, norm_A, W1, b1, W2, b2, alpha_params):
    raise NotImplementedError("write your pallas kernel here")


# speedup vs baseline: 2.6276x; 2.6276x over previous
import functools

import jax
import jax.numpy as jnp
from jax import lax
from jax.experimental import pallas as pl
from jax.experimental.pallas import tpu as pltpu
from jax.experimental.pallas import tpu_sc as plsc

_LANES = 16
_NUM_TILES = 32
_CHUNK = 4000


def _front_body(f_ref, w_ref, b_ref, n_ref, o_ref):
    x = lax.dot_general(w_ref[...], f_ref[...], (((0,), (1,)), ((), ())),
                        preferred_element_type=jnp.float32)
    x = jnp.maximum(x + b_ref[...], 0.0) + n_ref[...]
    nrm = jnp.sqrt(jnp.sum(x * x, axis=1, keepdims=True))
    o_ref[...] = x / jnp.maximum(nrm, 1e-8)


def _back_body(a_ref, w_ref, b_ref, o_ref):
    o_ref[...] = lax.dot_general(
        a_ref[...], w_ref[...], (((0,), (0,)), ((), ())),
        preferred_element_type=jnp.float32) + b_ref[...]


def _rsqrt_scalar(ss):
    i = lax.bitcast_convert_type(ss, jnp.int32)
    y = lax.bitcast_convert_type(
        jnp.int32(0x5F3759DF) - lax.shift_right_arithmetic(i, 1), jnp.float32)
    for _ in range(4):
        y = y * (1.5 - 0.5 * ss * y * y)
    return y


def _sc_hop_body(h0t, src_hbm, dst_hbm, nrm_hbm, alpha_hbm, out_hbm,
                 buf_a, buf_b, buf_c, acc, srcv, dstv, nrmv, alphav,
                 *, n_nodes, n_edges, n_hops):
    n = n_nodes
    cid = lax.axis_index("c")
    sid = lax.axis_index("s")
    ch0 = 2 * (sid * 2 + cid)
    nvec = n // _LANES

    pltpu.sync_copy(h0t.at[pl.ds(ch0 * n, 2 * n)], buf_a)
    pltpu.sync_copy(alpha_hbm.at[pl.ds(ch0 * _LANES, 2 * _LANES)], alphav)

    alpha_row0 = alphav[pl.ds(0, _LANES)]
    alpha_row1 = alphav[pl.ds(_LANES, _LANES)]
    a00 = alpha_row0[0]
    a10 = alpha_row1[0]

    def init_body(i, _):
        o = i * _LANES
        s0 = pl.ds(o, _LANES)
        s1 = pl.ds(n + o, _LANES)
        acc[s0] = a00 * buf_a[s0]
        acc[s1] = a10 * buf_a[s1]
        buf_b[s0] = jnp.zeros((_LANES,), jnp.float32)
        buf_b[s1] = jnp.zeros((_LANES,), jnp.float32)
        return 0

    lax.fori_loop(0, nvec, init_body, 0)

    zed = jnp.zeros((_LANES,), jnp.float32)
    nofs = jnp.full((_LANES,), n, jnp.int32)

    last, second, rst = buf_a, buf_b, buf_c
    for hop in range(1, n_hops + 1):
        def zero_body(i, _, rst=rst):
            o = i * _LANES
            rst[pl.ds(o, _LANES)] = zed
            rst[pl.ds(n + o, _LANES)] = zed
            return 0

        lax.fori_loop(0, nvec, zero_body, 0)

        def chunk_body(ci, _, last=last, rst=rst):
            base = ci * _CHUNK
            pltpu.sync_copy(src_hbm.at[pl.ds(base, _CHUNK)], srcv)
            pltpu.sync_copy(dst_hbm.at[pl.ds(base, _CHUNK)], dstv)
            pltpu.sync_copy(nrm_hbm.at[pl.ds(base, _CHUNK)], nrmv)

            def edge_body(k, _):
                sl = pl.ds(k * _LANES, _LANES)
                s = srcv[sl]
                d = dstv[sl]
                nm = nrmv[sl]
                v0 = plsc.load_gather(last, [s])
                plsc.addupdate_scatter(rst, [d], nm * v0)
                v1 = plsc.load_gather(last, [s + nofs])
                plsc.addupdate_scatter(rst, [d + nofs], nm * v1)
                return 0

            lax.fori_loop(0, _CHUNK // _LANES, edge_body, 0)
            return 0

        lax.fori_loop(0, n_edges // _CHUNK, chunk_body, 0)

        def dots_body(i, carry, last=last, second=second, rst=rst):
            rl0, rl1, rs0, rs1, ls0, ls1 = carry
            o = i * _LANES
            s0 = pl.ds(o, _LANES)
            s1 = pl.ds(n + o, _LANES)
            r0, r1 = rst[s0], rst[s1]
            l0, l1 = last[s0], last[s1]
            q0, q1 = second[s0], second[s1]
            return (rl0 + r0 * l0, rl1 + r1 * l1,
                    rs0 + r0 * q0, rs1 + r1 * q1,
                    ls0 + l0 * q0, ls1 + l1 * q1)

        rl0, rl1, rs0, rs1, ls0, ls1 = lax.fori_loop(
            0, nvec, dots_body, (zed, zed, zed, zed, zed, zed))
        t1_0, t1_1 = jnp.sum(rl0), jnp.sum(rl1)
        t2_0 = jnp.sum(rs0) - t1_0 * jnp.sum(ls0)
        t2_1 = jnp.sum(rs1) - t1_1 * jnp.sum(ls1)

        def ortho_body(i, carry, last=last, second=second, rst=rst):
            ss0, ss1 = carry
            o = i * _LANES
            s0 = pl.ds(o, _LANES)
            s1 = pl.ds(n + o, _LANES)
            v0 = rst[s0] - t1_0 * last[s0] - t2_0 * second[s0]
            v1 = rst[s1] - t1_1 * last[s1] - t2_1 * second[s1]
            rst[s0] = v0
            rst[s1] = v1
            return (ss0 + v0 * v0, ss1 + v1 * v1)

        ss0v, ss1v = lax.fori_loop(0, nvec, ortho_body, (zed, zed))
        ss0, ss1 = jnp.sum(ss0v), jnp.sum(ss1v)
        inv0 = jnp.where(ss0 < 1e-16, 1e8, _rsqrt_scalar(ss0))
        inv1 = jnp.where(ss1 < 1e-16, 1e8, _rsqrt_scalar(ss1))
        ai0 = alpha_row0[hop]
        ai1 = alpha_row1[hop]

        def scale_body(i, _, rst=rst):
            o = i * _LANES
            s0 = pl.ds(o, _LANES)
            s1 = pl.ds(n + o, _LANES)
            v0 = rst[s0] * inv0
            v1 = rst[s1] * inv1
            rst[s0] = v0
            rst[s1] = v1
            acc[s0] = acc[s0] + ai0 * v0
            acc[s1] = acc[s1] + ai1 * v1
            return 0

        lax.fori_loop(0, nvec, scale_body, 0)

        last, second, rst = rst, last, second

    pltpu.sync_copy(acc, out_hbm.at[pl.ds(ch0 * n, 2 * n)])


def kernel(features, edge_index, norm_A, W1, b1, W2, b2, alpha_params):
    n, in_feats = features.shape
    hidden = W1.shape[1]
    n_edges = norm_A.shape[0]
    n_hops = alpha_params.shape[1] - 1

    noise_t = (jax.random.normal(jax.random.key(42), (n, hidden),
                                 dtype=jnp.float32) * 1e-5).T
    b1_col = b1.reshape(hidden, 1)
    b2_row = b2.reshape(1, -1)
    alpha_pad = jnp.zeros((hidden, _LANES), jnp.float32)
    alpha_pad = alpha_pad.at[:, : n_hops + 1].set(alpha_params)
    alpha_flat = alpha_pad.reshape(-1)
    src = edge_index[0]
    dst = edge_index[1]

    h0t = pl.pallas_call(
        _front_body,
        out_shape=jax.ShapeDtypeStruct((hidden, n), jnp.float32),
    )(features, W1, b1_col, noise_t)
    h0t_flat = h0t.reshape(-1)

    mesh = plsc.VectorSubcoreMesh(core_axis_name="c", subcore_axis_name="s")
    sc_fn = pl.kernel(
        functools.partial(_sc_hop_body, n_nodes=n, n_edges=n_edges,
                          n_hops=n_hops),
        out_type=jax.ShapeDtypeStruct((hidden * n,), jnp.float32),
        mesh=mesh,
        compiler_params=pltpu.CompilerParams(needs_layout_passes=False),
        scratch_types=[
            pltpu.VMEM((2 * n,), jnp.float32),
            pltpu.VMEM((2 * n,), jnp.float32),
            pltpu.VMEM((2 * n,), jnp.float32),
            pltpu.VMEM((2 * n,), jnp.float32),
            pltpu.VMEM((_CHUNK,), jnp.int32),
            pltpu.VMEM((_CHUNK,), jnp.int32),
            pltpu.VMEM((_CHUNK,), jnp.float32),
            pltpu.VMEM((2 * _LANES,), jnp.float32),
        ],
    )
    acc_t = sc_fn(h0t_flat, src, dst, norm_A, alpha_flat).reshape(hidden, n)

    out = pl.pallas_call(
        _back_body,
        out_shape=jax.ShapeDtypeStruct((n, W2.shape[1]), jnp.float32),
    )(acc_t, W2, b2_row)
    return out


# speedup vs baseline: 10.7164x; 4.0784x over previous
import functools

import jax
import jax.numpy as jnp
from jax import lax
from jax.experimental import pallas as pl
from jax.experimental.pallas import tpu as pltpu
from jax.experimental.pallas import tpu_sc as plsc

_LANES = 16
_CHUNK = 2560
_IDXBITS = 14


def _front_body(f_ref, w_ref, b_ref, n_ref, o_ref):
    x = lax.dot_general(w_ref[...], f_ref[...], (((0,), (1,)), ((), ())),
                        preferred_element_type=jnp.float32)
    x = jnp.maximum(x + b_ref[...], 0.0) + n_ref[...]
    nrm = jnp.sqrt(jnp.sum(x * x, axis=1, keepdims=True))
    o_ref[...] = x / jnp.maximum(nrm, 1e-8)


def _back_body(a_ref, w_ref, b_ref, o_ref):
    o_ref[...] = lax.dot_general(
        a_ref[...], w_ref[...], (((0,), (0,)), ((), ())),
        preferred_element_type=jnp.float32) + b_ref[...]


def _rsqrt_scalar(ss):
    i = lax.bitcast_convert_type(ss, jnp.int32)
    y = lax.bitcast_convert_type(
        jnp.int32(0x5F3759DF) - lax.shift_right_arithmetic(i, 1), jnp.float32)
    for _ in range(4):
        y = y * (1.5 - 0.5 * ss * y * y)
    return y


def _sc_hop_body(h0t, pk_hbm, nrm_hbm, alpha_hbm, out_hbm,
                 buf_a, buf_b, buf_c, acc,
                 pk0, nrm0, pk1, nrm1, alphav, sem0, sem1,
                 *, n_nodes, n_edges, n_hops):
    n = n_nodes
    cid = lax.axis_index("c")
    sid = lax.axis_index("s")
    ch0 = 2 * (sid * 2 + cid)
    nvec = n // _LANES
    nchunks = n_edges // _CHUNK
    npairs = (nchunks - 1) // 2

    pltpu.sync_copy(h0t.at[pl.ds(ch0 * n, 2 * n)], buf_a)
    pltpu.sync_copy(alpha_hbm.at[pl.ds(ch0 * _LANES, 2 * _LANES)], alphav)

    alpha_row0 = alphav[pl.ds(0, _LANES)]
    alpha_row1 = alphav[pl.ds(_LANES, _LANES)]
    a00 = alpha_row0[0]
    a10 = alpha_row1[0]

    def init_body(i, _):
        o = i * _LANES
        s0 = pl.ds(o, _LANES)
        s1 = pl.ds(n + o, _LANES)
        z = jnp.zeros((_LANES,), jnp.float32)
        acc[s0] = a00 * buf_a[s0]
        acc[s1] = a10 * buf_a[s1]
        buf_b[s0] = z
        buf_b[s1] = z
        buf_c[s0] = z
        buf_c[s1] = z
        return 0

    lax.fori_loop(0, nvec, init_body, 0, unroll=4)

    zed = jnp.zeros((_LANES,), jnp.float32)
    nofs = jnp.full((_LANES,), n, jnp.int32)
    mask = jnp.full((_LANES,), (1 << _IDXBITS) - 1, jnp.int32)

    def start(ci, pkb, nrmb, sem):
        base = ci * _CHUNK
        pltpu.async_copy(pk_hbm.at[pl.ds(base, _CHUNK)], pkb, sem)
        pltpu.async_copy(nrm_hbm.at[pl.ds(base, _CHUNK)], nrmb, sem)

    def wait(pkb, nrmb, sem):
        pltpu.make_async_copy(pk_hbm.at[pl.ds(0, _CHUNK)], pkb, sem).wait()
        pltpu.make_async_copy(nrm_hbm.at[pl.ds(0, _CHUNK)], nrmb, sem).wait()

    last, second, rst = buf_a, buf_b, buf_c
    for hop in range(1, n_hops + 1):
        def compute(pkb, nrmb, last=last, rst=rst):
            @plsc.parallel_loop(0, _CHUNK // _LANES, unroll=8)
            def _(k):
                sl = pl.ds(k * _LANES, _LANES)
                pk = pkb[sl]
                nm = nrmb[sl]
                s = pk & mask
                d = lax.shift_right_logical(pk, _IDXBITS)
                v0 = plsc.load_gather(last, [s])
                plsc.addupdate_scatter(rst, [d], nm * v0)
                v1 = plsc.load_gather(last, [s + nofs])
                plsc.addupdate_scatter(rst, [d + nofs], nm * v1)

        start(0, pk0, nrm0, sem0)

        def pair_body(i, _, last=last, rst=rst):
            c0 = 2 * i
            start(c0 + 1, pk1, nrm1, sem1)
            wait(pk0, nrm0, sem0)
            compute(pk0, nrm0, last=last, rst=rst)
            start(c0 + 2, pk0, nrm0, sem0)
            wait(pk1, nrm1, sem1)
            compute(pk1, nrm1, last=last, rst=rst)
            return 0

        lax.fori_loop(0, npairs, pair_body, 0)
        wait(pk0, nrm0, sem0)
        compute(pk0, nrm0, last=last, rst=rst)

        def dots_body(i, carry, last=last, second=second, rst=rst):
            rl0, rl1, rs0, rs1, ls0, ls1 = carry
            o = i * _LANES
            s0 = pl.ds(o, _LANES)
            s1 = pl.ds(n + o, _LANES)
            r0, r1 = rst[s0], rst[s1]
            l0, l1 = last[s0], last[s1]
            q0, q1 = second[s0], second[s1]
            return (rl0 + r0 * l0, rl1 + r1 * l1,
                    rs0 + r0 * q0, rs1 + r1 * q1,
                    ls0 + l0 * q0, ls1 + l1 * q1)

        rl0, rl1, rs0, rs1, ls0, ls1 = lax.fori_loop(
            0, nvec, dots_body, (zed, zed, zed, zed, zed, zed), unroll=4)
        t1_0, t1_1 = jnp.sum(rl0), jnp.sum(rl1)
        t2_0 = jnp.sum(rs0) - t1_0 * jnp.sum(ls0)
        t2_1 = jnp.sum(rs1) - t1_1 * jnp.sum(ls1)

        def ortho_body(i, carry, last=last, second=second, rst=rst):
            ss0, ss1 = carry
            o = i * _LANES
            s0 = pl.ds(o, _LANES)
            s1 = pl.ds(n + o, _LANES)
            v0 = rst[s0] - t1_0 * last[s0] - t2_0 * second[s0]
            v1 = rst[s1] - t1_1 * last[s1] - t2_1 * second[s1]
            rst[s0] = v0
            rst[s1] = v1
            return (ss0 + v0 * v0, ss1 + v1 * v1)

        ss0v, ss1v = lax.fori_loop(0, nvec, ortho_body, (zed, zed), unroll=4)
        ss0, ss1 = jnp.sum(ss0v), jnp.sum(ss1v)
        inv0 = jnp.where(ss0 < 1e-16, 1e8, _rsqrt_scalar(ss0))
        inv1 = jnp.where(ss1 < 1e-16, 1e8, _rsqrt_scalar(ss1))
        ai0 = alpha_row0[hop]
        ai1 = alpha_row1[hop]

        def scale_body(i, _, second=second, rst=rst):
            o = i * _LANES
            s0 = pl.ds(o, _LANES)
            s1 = pl.ds(n + o, _LANES)
            v0 = rst[s0] * inv0
            v1 = rst[s1] * inv1
            rst[s0] = v0
            rst[s1] = v1
            acc[s0] = acc[s0] + ai0 * v0
            acc[s1] = acc[s1] + ai1 * v1
            if hop < n_hops:
                second[s0] = zed
                second[s1] = zed
            return 0

        lax.fori_loop(0, nvec, scale_body, 0, unroll=4)

        last, second, rst = rst, last, second

    pltpu.sync_copy(acc, out_hbm.at[pl.ds(ch0 * n, 2 * n)])


def kernel(features, edge_index, norm_A, W1, b1, W2, b2, alpha_params):
    n, in_feats = features.shape
    hidden = W1.shape[1]
    n_edges = norm_A.shape[0]
    n_hops = alpha_params.shape[1] - 1

    noise_t = (jax.random.normal(jax.random.key(42), (n, hidden),
                                 dtype=jnp.float32) * 1e-5).T
    b1_col = b1.reshape(hidden, 1)
    b2_row = b2.reshape(1, -1)
    alpha_pad = jnp.zeros((hidden, _LANES), jnp.float32)
    alpha_pad = alpha_pad.at[:, : n_hops + 1].set(alpha_params)
    alpha_flat = alpha_pad.reshape(-1)
    packed = edge_index[0] | (edge_index[1] << _IDXBITS)

    h0t = pl.pallas_call(
        _front_body,
        out_shape=jax.ShapeDtypeStruct((hidden, n), jnp.float32),
    )(features, W1, b1_col, noise_t)
    h0t_flat = h0t.reshape(-1)

    mesh = plsc.VectorSubcoreMesh(core_axis_name="c", subcore_axis_name="s")
    sc_fn = pl.kernel(
        functools.partial(_sc_hop_body, n_nodes=n, n_edges=n_edges,
                          n_hops=n_hops),
        out_type=jax.ShapeDtypeStruct((hidden * n,), jnp.float32),
        mesh=mesh,
        compiler_params=pltpu.CompilerParams(needs_layout_passes=False),
        scratch_types=[
            pltpu.VMEM((2 * n,), jnp.float32),
            pltpu.VMEM((2 * n,), jnp.float32),
            pltpu.VMEM((2 * n,), jnp.float32),
            pltpu.VMEM((2 * n,), jnp.float32),
            pltpu.VMEM((_CHUNK,), jnp.int32),
            pltpu.VMEM((_CHUNK,), jnp.float32),
            pltpu.VMEM((_CHUNK,), jnp.int32),
            pltpu.VMEM((_CHUNK,), jnp.float32),
            pltpu.VMEM((2 * _LANES,), jnp.float32),
            pltpu.SemaphoreType.DMA,
            pltpu.SemaphoreType.DMA,
        ],
    )
    acc_t = sc_fn(h0t_flat, packed, norm_A, alpha_flat).reshape(hidden, n)

    out = pl.pallas_call(
        _back_body,
        out_shape=jax.ShapeDtypeStruct((n, W2.shape[1]), jnp.float32),
    )(acc_t, W2, b2_row)
    return out


# speedup vs baseline: 10.8592x; 1.0133x over previous
import functools

import jax
import jax.numpy as jnp
from jax import lax
from jax.experimental import pallas as pl
from jax.experimental.pallas import tpu as pltpu
from jax.experimental.pallas import tpu_sc as plsc

_LANES = 16
_CHUNK = 2560
_IDXBITS = 14


def _front_body(f_ref, w_ref, b_ref, n_ref, o_ref):
    x = lax.dot_general(w_ref[...], f_ref[...], (((0,), (1,)), ((), ())),
                        preferred_element_type=jnp.float32)
    x = jnp.maximum(x + b_ref[...], 0.0) + n_ref[...]
    nrm = jnp.sqrt(jnp.sum(x * x, axis=1, keepdims=True))
    o_ref[...] = x / jnp.maximum(nrm, 1e-8)


def _back_body(a_ref, w_ref, b_ref, o_ref):
    o_ref[...] = lax.dot_general(
        a_ref[...], w_ref[...], (((0,), (0,)), ((), ())),
        preferred_element_type=jnp.float32) + b_ref[...]


def _rsqrt_scalar(ss):
    i = lax.bitcast_convert_type(ss, jnp.int32)
    y = lax.bitcast_convert_type(
        jnp.int32(0x5F3759DF) - lax.shift_right_arithmetic(i, 1), jnp.float32)
    for _ in range(4):
        y = y * (1.5 - 0.5 * ss * y * y)
    return y


def _sc_hop_body(h0t, pk_hbm, nrm_hbm, alpha_hbm, out_hbm,
                 buf_a, buf_b, buf_c, acc,
                 pk0, nrm0, pk1, nrm1, alphav, spk, snm, sem0, sem1,
                 *, n_nodes, n_edges, n_hops):
    n = n_nodes
    cid = lax.axis_index("c")
    sid = lax.axis_index("s")
    ch0 = 2 * (sid * 2 + cid)
    nvec = n // _LANES
    nchunks = n_edges // _CHUNK
    npairs = (nchunks - 1) // 2

    @pl.when(sid == 0)
    def _stage():
        pltpu.sync_copy(pk_hbm, spk)
        pltpu.sync_copy(nrm_hbm, snm)

    pltpu.sync_copy(h0t.at[pl.ds(ch0 * n, 2 * n)], buf_a)
    pltpu.sync_copy(alpha_hbm.at[pl.ds(ch0 * _LANES, 2 * _LANES)], alphav)
    plsc.subcore_barrier()

    alpha_row0 = alphav[pl.ds(0, _LANES)]
    alpha_row1 = alphav[pl.ds(_LANES, _LANES)]
    a00 = alpha_row0[0]
    a10 = alpha_row1[0]

    def init_body(i, _):
        o = i * _LANES
        s0 = pl.ds(o, _LANES)
        s1 = pl.ds(n + o, _LANES)
        z = jnp.zeros((_LANES,), jnp.float32)
        acc[s0] = a00 * buf_a[s0]
        acc[s1] = a10 * buf_a[s1]
        buf_b[s0] = z
        buf_b[s1] = z
        buf_c[s0] = z
        buf_c[s1] = z
        return 0

    lax.fori_loop(0, nvec, init_body, 0, unroll=4)

    zed = jnp.zeros((_LANES,), jnp.float32)
    nofs = jnp.full((_LANES,), n, jnp.int32)
    mask = jnp.full((_LANES,), (1 << _IDXBITS) - 1, jnp.int32)

    def start(ci, pkb, nrmb, sem):
        base = ci * _CHUNK
        pltpu.async_copy(spk.at[pl.ds(base, _CHUNK)], pkb, sem)
        pltpu.async_copy(snm.at[pl.ds(base, _CHUNK)], nrmb, sem)

    def wait(pkb, nrmb, sem):
        pltpu.make_async_copy(spk.at[pl.ds(0, _CHUNK)], pkb, sem).wait()
        pltpu.make_async_copy(snm.at[pl.ds(0, _CHUNK)], nrmb, sem).wait()

    last, second, rst = buf_a, buf_b, buf_c
    for hop in range(1, n_hops + 1):
        def compute(pkb, nrmb, last=last, rst=rst):
            @plsc.parallel_loop(0, _CHUNK // _LANES, unroll=8)
            def _(k):
                sl = pl.ds(k * _LANES, _LANES)
                pk = pkb[sl]
                nm = nrmb[sl]
                s = pk & mask
                d = lax.shift_right_logical(pk, _IDXBITS)
                v0 = plsc.load_gather(last, [s])
                plsc.addupdate_scatter(rst, [d], nm * v0)
                v1 = plsc.load_gather(last, [s + nofs])
                plsc.addupdate_scatter(rst, [d + nofs], nm * v1)

        start(0, pk0, nrm0, sem0)

        def pair_body(i, _, last=last, rst=rst):
            c0 = 2 * i
            start(c0 + 1, pk1, nrm1, sem1)
            wait(pk0, nrm0, sem0)
            compute(pk0, nrm0, last=last, rst=rst)
            start(c0 + 2, pk0, nrm0, sem0)
            wait(pk1, nrm1, sem1)
            compute(pk1, nrm1, last=last, rst=rst)
            return 0

        lax.fori_loop(0, npairs, pair_body, 0)
        wait(pk0, nrm0, sem0)
        compute(pk0, nrm0, last=last, rst=rst)

        def dots_body(i, carry, last=last, second=second, rst=rst):
            rl0, rl1, rs0, rs1, ls0, ls1 = carry
            o = i * _LANES
            s0 = pl.ds(o, _LANES)
            s1 = pl.ds(n + o, _LANES)
            r0, r1 = rst[s0], rst[s1]
            l0, l1 = last[s0], last[s1]
            q0, q1 = second[s0], second[s1]
            return (rl0 + r0 * l0, rl1 + r1 * l1,
                    rs0 + r0 * q0, rs1 + r1 * q1,
                    ls0 + l0 * q0, ls1 + l1 * q1)

        rl0, rl1, rs0, rs1, ls0, ls1 = lax.fori_loop(
            0, nvec, dots_body, (zed, zed, zed, zed, zed, zed), unroll=4)
        t1_0, t1_1 = jnp.sum(rl0), jnp.sum(rl1)
        t2_0 = jnp.sum(rs0) - t1_0 * jnp.sum(ls0)
        t2_1 = jnp.sum(rs1) - t1_1 * jnp.sum(ls1)

        def ortho_body(i, carry, last=last, second=second, rst=rst):
            ss0, ss1 = carry
            o = i * _LANES
            s0 = pl.ds(o, _LANES)
            s1 = pl.ds(n + o, _LANES)
            v0 = rst[s0] - t1_0 * last[s0] - t2_0 * second[s0]
            v1 = rst[s1] - t1_1 * last[s1] - t2_1 * second[s1]
            rst[s0] = v0
            rst[s1] = v1
            return (ss0 + v0 * v0, ss1 + v1 * v1)

        ss0v, ss1v = lax.fori_loop(0, nvec, ortho_body, (zed, zed), unroll=4)
        ss0, ss1 = jnp.sum(ss0v), jnp.sum(ss1v)
        inv0 = jnp.where(ss0 < 1e-16, 1e8, _rsqrt_scalar(ss0))
        inv1 = jnp.where(ss1 < 1e-16, 1e8, _rsqrt_scalar(ss1))
        ai0 = alpha_row0[hop]
        ai1 = alpha_row1[hop]

        def scale_body(i, _, second=second, rst=rst):
            o = i * _LANES
            s0 = pl.ds(o, _LANES)
            s1 = pl.ds(n + o, _LANES)
            v0 = rst[s0] * inv0
            v1 = rst[s1] * inv1
            rst[s0] = v0
            rst[s1] = v1
            acc[s0] = acc[s0] + ai0 * v0
            acc[s1] = acc[s1] + ai1 * v1
            if hop < n_hops:
                second[s0] = zed
                second[s1] = zed
            return 0

        lax.fori_loop(0, nvec, scale_body, 0, unroll=4)

        last, second, rst = rst, last, second

    pltpu.sync_copy(acc, out_hbm.at[pl.ds(ch0 * n, 2 * n)])


def kernel(features, edge_index, norm_A, W1, b1, W2, b2, alpha_params):
    n, in_feats = features.shape
    hidden = W1.shape[1]
    n_edges = norm_A.shape[0]
    n_hops = alpha_params.shape[1] - 1

    noise_t = (jax.random.normal(jax.random.key(42), (n, hidden),
                                 dtype=jnp.float32) * 1e-5).T
    b1_col = b1.reshape(hidden, 1)
    b2_row = b2.reshape(1, -1)
    alpha_pad = jnp.zeros((hidden, _LANES), jnp.float32)
    alpha_pad = alpha_pad.at[:, : n_hops + 1].set(alpha_params)
    alpha_flat = alpha_pad.reshape(-1)
    packed = edge_index[0] | (edge_index[1] << _IDXBITS)

    h0t = pl.pallas_call(
        _front_body,
        out_shape=jax.ShapeDtypeStruct((hidden, n), jnp.float32),
    )(features, W1, b1_col, noise_t)
    h0t_flat = h0t.reshape(-1)

    mesh = plsc.VectorSubcoreMesh(core_axis_name="c", subcore_axis_name="s")
    sc_fn = pl.kernel(
        functools.partial(_sc_hop_body, n_nodes=n, n_edges=n_edges,
                          n_hops=n_hops),
        out_type=jax.ShapeDtypeStruct((hidden * n,), jnp.float32),
        mesh=mesh,
        compiler_params=pltpu.CompilerParams(needs_layout_passes=False),
        scratch_types=[
            pltpu.VMEM((2 * n,), jnp.float32),
            pltpu.VMEM((2 * n,), jnp.float32),
            pltpu.VMEM((2 * n,), jnp.float32),
            pltpu.VMEM((2 * n,), jnp.float32),
            pltpu.VMEM((_CHUNK,), jnp.int32),
            pltpu.VMEM((_CHUNK,), jnp.float32),
            pltpu.VMEM((_CHUNK,), jnp.int32),
            pltpu.VMEM((_CHUNK,), jnp.float32),
            pltpu.VMEM((2 * _LANES,), jnp.float32),
            pltpu.VMEM_SHARED((n_edges,), jnp.int32),
            pltpu.VMEM_SHARED((n_edges,), jnp.float32),
            pltpu.SemaphoreType.DMA,
            pltpu.SemaphoreType.DMA,
        ],
    )
    acc_t = sc_fn(h0t_flat, packed, norm_A, alpha_flat).reshape(hidden, n)

    out = pl.pallas_call(
        _back_body,
        out_shape=jax.ShapeDtypeStruct((n, W2.shape[1]), jnp.float32),
    )(acc_t, W2, b2_row)
    return out
